# baseline (speedup 1.0000x reference)
import functools

import jax
import jax.numpy as jnp
from jax import lax
from jax.experimental import pallas as pl
from jax.experimental.pallas import tpu as pltpu
from jax.experimental.pallas import tpu_sc as plsc


def kernel(input_seq, weights):
    batch, seq_len = input_seq.shape
    _, emb = weights.shape

    info = plsc.get_sparse_core_info()
    num_workers = info.num_cores * info.num_subcores
    rows_per_w = seq_len // num_workers

    mesh = plsc.VectorSubcoreMesh(core_axis_name="c", subcore_axis_name="s")

    @functools.partial(
        pl.kernel,
        out_type=jax.ShapeDtypeStruct((batch, seq_len, emb), weights.dtype),
        mesh=mesh,
        scratch_types=[pltpu.SemaphoreType.DMA],
    )
    def _bcast(weights_hbm, out_hbm, sem):
        wid = lax.axis_index("s") * info.num_cores + lax.axis_index("c")
        base = wid * rows_per_w
        src = weights_hbm.at[pl.ds(base, rows_per_w)]
        copies = [
            pltpu.make_async_copy(src, out_hbm.at[b, pl.ds(base, rows_per_w)], sem)
            for b in range(batch)
        ]
        for c in copies:
            c.start()
        for c in copies:
            c.wait()

    return _bcast(weights)


# speedup vs baseline: 77.6235x; 77.6235x over previous
import jax
import jax.numpy as jnp
from jax.experimental import pallas as pl


_ROWS = 512


def _bcast_body(w_ref, o_ref):
    o_ref[...] = jnp.broadcast_to(w_ref[...][None], o_ref.shape)


def kernel(input_seq, weights):
    batch, seq_len = input_seq.shape
    _, emb = weights.shape
    n_blocks = seq_len // _ROWS

    return pl.pallas_call(
        _bcast_body,
        grid=(n_blocks,),
        in_specs=[pl.BlockSpec((_ROWS, emb), lambda i: (i, 0))],
        out_specs=pl.BlockSpec((batch, _ROWS, emb), lambda i: (0, i, 0)),
        out_shape=jax.ShapeDtypeStruct((batch, seq_len, emb), weights.dtype),
    )(weights)
